# baseline (speedup 1.0000x reference)
import jax
import jax.numpy as jnp
from jax.experimental import pallas as pl
from jax.experimental.pallas import tpu as pltpu

_SPLIT = 8
_SUB = 5000


def _fused_mlp(*refs):
    x_refs = refs[:_SPLIT]
    w1_ref, b1_ref, w2_ref, b2_ref, o_ref = refs[_SPLIT:]
    w1 = w1_ref[...]
    b1 = b1_ref[...]
    w2 = w2_ref[...]
    b2 = b2_ref[0, 0]
    for j in range(_SPLIT):
        x = x_refs[j][...]
        hT = jax.lax.dot_general(
            w1, x,
            dimension_numbers=(((1,), (1,)), ((), ())),
            preferred_element_type=jnp.float32,
        )
        hT = jnp.maximum(hT + b1, 0.0)
        y = jax.lax.dot_general(
            w2, hT,
            dimension_numbers=(((1,), (0,)), ((), ())),
            preferred_element_type=jnp.float32,
        )
        o_ref[0, :, j * _SUB:(j + 1) * _SUB] = y + b2


def kernel(embeddings, W1, b1, W2, b2):
    n, d = embeddings.shape
    hdim = W1.shape[0]
    b1r = b1.reshape(hdim, 1)
    b2r = b2.reshape(1, 1)
    step = _SPLIT * _SUB
    nb = n // step
    x_specs = [
        pl.BlockSpec((_SUB, d), lambda i, j=j: (i * _SPLIT + j, 0))
        for j in range(_SPLIT)
    ]
    out = pl.pallas_call(
        _fused_mlp,
        grid=(nb,),
        in_specs=x_specs + [
            pl.BlockSpec((hdim, d), lambda i: (0, 0)),
            pl.BlockSpec((hdim, 1), lambda i: (0, 0)),
            pl.BlockSpec((1, hdim), lambda i: (0, 0)),
            pl.BlockSpec((1, 1), lambda i: (0, 0)),
        ],
        out_specs=pl.BlockSpec((1, 1, step), lambda i: (i, 0, 0)),
        out_shape=jax.ShapeDtypeStruct((nb, 1, step), jnp.float32),
        compiler_params=pltpu.CompilerParams(
            dimension_semantics=("arbitrary",),
        ),
    )(*([embeddings] * _SPLIT), W1, b1r, W2, b2r)
    return out.reshape(n)
